# baseline (speedup 1.0000x reference)
import functools

import jax
import jax.numpy as jnp
from jax import lax
from jax.experimental import pallas as pl
from jax.experimental.pallas import tpu as pltpu
from jax.experimental.pallas import tpu_sc as plsc

N = 10000
E = 320000
D = 128
H = 128
G = 64
C = 10

NC = 2
NS = 16
NW = NC * NS

NR = 10240
DUMMY_ROW = 10000
DUMMY_SEG = G

EK = 128
NCHUNK = 80
EPW = NCHUNK * EK
E_PAD = NW * EPW

RPT = NR // NS
RPW = NR // NW
GP = G + 1
CNT_PAD = 80

_mesh = plsc.VectorSubcoreMesh(core_axis_name="c", subcore_axis_name="s",
                               num_cores=NC, num_subcores=NS)


def _edge_agg_body(x_hbm, src_hbm, dst_hbm, out_hbm, sidx, didx, rows, aggs,
                   gsem):
    c = lax.axis_index("c")
    s = lax.axis_index("s")
    w = c * NS + s

    def _zero(i, _):
        r = i // (D // 16)
        g = i % (D // 16)
        rows[r, pl.ds(g * 16, 16)] = jnp.zeros((16,), jnp.float32)
        return 0

    lax.fori_loop(0, EK * (D // 16), _zero, 0)
    for k in range(RPT // EK):
        pltpu.sync_copy(rows, aggs.at[pl.ds(s * RPT + k * EK, EK)])
    plsc.subcore_barrier()

    def _step(j, _):
        row = w * NCHUNK + j
        pltpu.sync_copy(src_hbm.at[row], sidx)
        pltpu.sync_copy(dst_hbm.at[row], didx)
        pltpu.async_copy(x_hbm.at[sidx], rows, gsem).wait()
        pltpu.sync_copy(rows, aggs.at[didx], add=True)
        return 0

    lax.fori_loop(0, NCHUNK, _step, 0)

    plsc.subcore_barrier()
    pltpu.sync_copy(aggs.at[pl.ds(s * RPT, RPT)],
                    out_hbm.at[c, pl.ds(s * RPT, RPT)])


@functools.partial(
    pl.kernel,
    mesh=_mesh,
    out_type=jax.ShapeDtypeStruct((NC, NR, D), jnp.float32),
    scratch_types=[
        pltpu.VMEM((EK,), jnp.int32),
        pltpu.VMEM((EK,), jnp.int32),
        pltpu.VMEM((EK, D), jnp.float32),
        pltpu.VMEM_SHARED((NR, D), jnp.float32),
        pltpu.SemaphoreType.DMA,
    ],
)
def _edge_agg(x_hbm, src_hbm, dst_hbm, out_hbm, sidx, didx, rows, aggs, gsem):
    _edge_agg_body(x_hbm, src_hbm, dst_hbm, out_hbm, sidx, didx, rows, aggs,
                   gsem)


def _readout_body(h_hbm, batch_hbm, osum, omax, ocnt, bidx, hrows, accs, accm,
                  cnt):
    c = lax.axis_index("c")
    s = lax.axis_index("s")
    w = c * NS + s

    def _init(i, _):
        accs[pl.ds(i * 16, 16)] = jnp.zeros((16,), jnp.float32)
        accm[pl.ds(i * 16, 16)] = jnp.full((16,), -jnp.inf, jnp.float32)
        return 0

    lax.fori_loop(0, GP * D // 16, _init, 0)
    for k in range(CNT_PAD // 16):
        cnt[pl.ds(k * 16, 16)] = jnp.zeros((16,), jnp.float32)

    pltpu.sync_copy(batch_hbm.at[pl.ds(w * RPW, RPW)], bidx.at[pl.ds(0, RPW)])
    pltpu.sync_copy(h_hbm.at[pl.ds(w * RPW, RPW)], hrows)

    e0 = jnp.where(lax.iota(jnp.int32, 16) == 0, 1.0, 0.0).astype(jnp.float32)

    def _row(i, _):
        b = bidx[pl.ds(i, 16)][0]
        base = b * D
        for g in range(D // 16):
            r = hrows[i, pl.ds(g * 16, 16)]
            off = base + g * 16
            accs[pl.ds(off, 16)] = accs[pl.ds(off, 16)] + r
            accm[pl.ds(off, 16)] = jnp.maximum(accm[pl.ds(off, 16)], r)
        cnt[pl.ds(b, 16)] = cnt[pl.ds(b, 16)] + e0
        return 0

    lax.fori_loop(0, RPW, _row, 0)
    pltpu.sync_copy(accs, osum.at[w])
    pltpu.sync_copy(accm, omax.at[w])
    pltpu.sync_copy(cnt, ocnt.at[w])


@functools.partial(
    pl.kernel,
    mesh=_mesh,
    out_type=[
        jax.ShapeDtypeStruct((NW, GP * D), jnp.float32),
        jax.ShapeDtypeStruct((NW, GP * D), jnp.float32),
        jax.ShapeDtypeStruct((NW, CNT_PAD), jnp.float32),
    ],
    scratch_types=[
        pltpu.VMEM((RPW + 16,), jnp.int32),
        pltpu.VMEM((RPW, D), jnp.float32),
        pltpu.VMEM((GP * D,), jnp.float32),
        pltpu.VMEM((GP * D,), jnp.float32),
        pltpu.VMEM((CNT_PAD,), jnp.float32),
    ],
)
def _readout(h_hbm, batch_hbm, osum, omax, ocnt, bidx, hrows, accs, accm, cnt):
    _readout_body(h_hbm, batch_hbm, osum, omax, ocnt, bidx, hrows, accs, accm,
                  cnt)


def _mlp_body(x_ref, p0_ref, p1_ref, w1_ref, b1_ref, w2_ref, b2_ref, o_ref):
    h = x_ref[...] + p0_ref[...] + p1_ref[...]
    a = jnp.maximum(
        jnp.dot(h, w1_ref[...], preferred_element_type=jnp.float32)
        + b1_ref[...], 0.0)
    o = jnp.maximum(
        jnp.dot(a, w2_ref[...], preferred_element_type=jnp.float32)
        + b2_ref[...], 0.0)
    o_ref[...] = o


_MLP_BLK = 1024


def _mlp(x, p0, p1, W1, b1, W2, b2):
    row_spec = pl.BlockSpec((_MLP_BLK, D), lambda i: (i, 0))
    full = lambda a, b: pl.BlockSpec((a, b), lambda i: (0, 0))
    return pl.pallas_call(
        _mlp_body,
        grid=(NR // _MLP_BLK,),
        in_specs=[
            row_spec, row_spec, row_spec,
            full(D, H), full(1, H), full(H, H), full(1, H),
        ],
        out_specs=row_spec,
        out_shape=jax.ShapeDtypeStruct((NR, D), jnp.float32),
    )(x, p0, p1, W1, b1, W2, b2)


def _final_body(s2, m2, s3, m3, s4, m4, cn, wl1, bl1, wl2, bl2, o_ref):
    c = jnp.sum(cn[...], axis=1, keepdims=True)
    rc = 1.0 / jnp.maximum(c, 1.0)

    def block(s_ref, m_ref):
        ssum = jnp.sum(s_ref[...], axis=0)
        mx = jnp.max(m_ref[...], axis=0)
        mx = jnp.where(jnp.isfinite(mx), mx, 0.0)
        return ssum * rc, mx, ssum

    mean2, mx2, ss2 = block(s2, m2)
    mean3, mx3, ss3 = block(s3, m3)
    mean4, mx4, ss4 = block(s4, m4)
    g = jnp.concatenate(
        [mean2 + mean3 + mean4, mx2 + mx3 + mx4, ss2 + ss3 + ss4], axis=-1)
    z = jnp.maximum(
        jnp.dot(g, wl1[...], preferred_element_type=jnp.float32) + bl1[...],
        0.0)
    logits = jnp.dot(z, wl2[...], preferred_element_type=jnp.float32) + bl2[...]
    mm = jnp.max(logits, axis=-1, keepdims=True)
    e = jnp.exp(logits - mm)
    o_ref[...] = (logits - mm) - jnp.log(jnp.sum(e, axis=-1, keepdims=True))


def _final(s2, m2, s3, m3, s4, m4, cnt_t, Wl1, bl1, Wl2, bl2):
    return pl.pallas_call(
        _final_body,
        out_shape=jax.ShapeDtypeStruct((G, C), jnp.float32),
    )(s2, m2, s3, m3, s4, m4, cnt_t, Wl1, bl1, Wl2, bl2)


def kernel(x, edge_index, batch, W1a, b1a, W2a, b2a, W1b, b1b, W2b, b2b, Wl1,
           bl1, Wl2, bl2):
    src = edge_index[0]
    dst = edge_index[1]
    pad_e = E_PAD - E
    src_p = jnp.concatenate([src, jnp.zeros((pad_e,), jnp.int32)])
    dummy_dst = DUMMY_ROW + (jnp.arange(pad_e, dtype=jnp.int32) % (NR - N))
    dst_p = jnp.concatenate([dst, dummy_dst])
    src_p = src_p.reshape(NW * NCHUNK, EK)
    dst_p = dst_p.reshape(NW * NCHUNK, EK)
    x_p = jnp.concatenate([x, jnp.zeros((NR - N, D), jnp.float32)])
    batch_p = jnp.concatenate(
        [batch, jnp.full((NR - N,), DUMMY_SEG, jnp.int32)])

    b1a_r = b1a.reshape(1, H)
    b2a_r = b2a.reshape(1, H)
    b1b_r = b1b.reshape(1, H)
    b2b_r = b2b.reshape(1, H)
    bl1_r = bl1.reshape(1, H)
    bl2_r = bl2.reshape(1, C)

    def conv(h, W1, b1, W2, b2):
        parts = _edge_agg(h, src_p, dst_p)
        return _mlp(h, parts[0], parts[1], W1, b1, W2, b2)

    h1 = conv(x_p, W1a, b1a_r, W2a, b2a_r)
    h2 = conv(h1, W1b, b1b_r, W2b, b2b_r)
    s2, m2, c2 = _readout(h2, batch_p)
    h3 = conv(h2, W1b, b1b_r, W2b, b2b_r)
    s3, m3, c3 = _readout(h3, batch_p)
    h4 = conv(h3, W1b, b1b_r, W2b, b2b_r)
    s4, m4, c4 = _readout(h4, batch_p)

    rs = lambda a: a.reshape(NW, GP, D)[:, :G, :]
    cnt_t = c2[:, :G].T
    return _final(rs(s2), rs(m2), rs(s3), rs(m3), rs(s4), rs(m4), cnt_t, Wl1,
                  bl1_r, Wl2, bl2_r)


# speedup vs baseline: 1.0507x; 1.0507x over previous
import functools

import jax
import jax.numpy as jnp
from jax import lax
from jax.experimental import pallas as pl
from jax.experimental.pallas import tpu as pltpu
from jax.experimental.pallas import tpu_sc as plsc

N = 10000
E = 320000
D = 128
H = 128
G = 64
C = 10

NC = 2
NS = 16
NW = NC * NS

NR = 10240
DUMMY_ROW = 10000
DUMMY_SEG = G

EK = 128
NCHUNK = 80
EPW = NCHUNK * EK
E_PAD = NW * EPW

RPT = NR // NS
RPW = NR // NW
GP = G + 1
CNT_PAD = 80

_mesh = plsc.VectorSubcoreMesh(core_axis_name="c", subcore_axis_name="s",
                               num_cores=NC, num_subcores=NS)


def _edge_agg_body(x_hbm, src_hbm, dst_hbm, out_hbm, sidx, didx, rows, aggs,
                   gsem):
    c = lax.axis_index("c")
    s = lax.axis_index("s")
    w = c * NS + s

    def _zero(i, _):
        r = i // (D // 16)
        g = i % (D // 16)
        rows[r, pl.ds(g * 16, 16)] = jnp.zeros((16,), jnp.float32)
        return 0

    lax.fori_loop(0, EK * (D // 16), _zero, 0)
    for k in range(RPT // EK):
        pltpu.sync_copy(rows, aggs.at[pl.ds(s * RPT + k * EK, EK)])
    plsc.subcore_barrier()

    def _step(j, _):
        base = (w * NCHUNK + j) * EK
        pltpu.sync_copy(src_hbm.at[pl.ds(base, EK)], sidx)
        pltpu.sync_copy(dst_hbm.at[pl.ds(base, EK)], didx)
        pltpu.async_copy(x_hbm.at[sidx], rows, gsem).wait()
        pltpu.sync_copy(rows, aggs.at[didx], add=True)
        return 0

    lax.fori_loop(0, NCHUNK, _step, 0)

    plsc.subcore_barrier()
    pltpu.sync_copy(aggs.at[pl.ds(s * RPT, RPT)],
                    out_hbm.at[c, pl.ds(s * RPT, RPT)])


@functools.partial(
    pl.kernel,
    mesh=_mesh,
    out_type=jax.ShapeDtypeStruct((NC, NR, D), jnp.float32),
    scratch_types=[
        pltpu.VMEM((EK,), jnp.int32),
        pltpu.VMEM((EK,), jnp.int32),
        pltpu.VMEM((EK, D), jnp.float32),
        pltpu.VMEM_SHARED((NR, D), jnp.float32),
        pltpu.SemaphoreType.DMA,
    ],
)
def _edge_agg(x_hbm, src_hbm, dst_hbm, out_hbm, sidx, didx, rows, aggs, gsem):
    _edge_agg_body(x_hbm, src_hbm, dst_hbm, out_hbm, sidx, didx, rows, aggs,
                   gsem)


def _readout_body(h_hbm, batch_hbm, osum, omax, ocnt, bidx, hrows, accs, accm,
                  cnt):
    c = lax.axis_index("c")
    s = lax.axis_index("s")
    w = c * NS + s

    def _init(i, _):
        accs[pl.ds(i * 16, 16)] = jnp.zeros((16,), jnp.float32)
        accm[pl.ds(i * 16, 16)] = jnp.full((16,), -jnp.inf, jnp.float32)
        return 0

    lax.fori_loop(0, GP * D // 16, _init, 0)
    for k in range(CNT_PAD // 16):
        cnt[pl.ds(k * 16, 16)] = jnp.zeros((16,), jnp.float32)

    pltpu.sync_copy(batch_hbm.at[pl.ds(w * RPW, RPW)], bidx.at[pl.ds(0, RPW)])
    pltpu.sync_copy(h_hbm.at[pl.ds(w * RPW, RPW)], hrows)

    e0 = jnp.where(lax.iota(jnp.int32, 16) == 0, 1.0, 0.0).astype(jnp.float32)

    def _row(i, _):
        b = bidx[pl.ds(i, 16)][0]
        base = b * D
        for g in range(D // 16):
            r = hrows[i, pl.ds(g * 16, 16)]
            off = base + g * 16
            accs[pl.ds(off, 16)] = accs[pl.ds(off, 16)] + r
            accm[pl.ds(off, 16)] = jnp.maximum(accm[pl.ds(off, 16)], r)
        cnt[pl.ds(b, 16)] = cnt[pl.ds(b, 16)] + e0
        return 0

    lax.fori_loop(0, RPW, _row, 0)
    pltpu.sync_copy(accs, osum.at[w])
    pltpu.sync_copy(accm, omax.at[w])
    pltpu.sync_copy(cnt, ocnt.at[w])


@functools.partial(
    pl.kernel,
    mesh=_mesh,
    out_type=[
        jax.ShapeDtypeStruct((NW, GP * D), jnp.float32),
        jax.ShapeDtypeStruct((NW, GP * D), jnp.float32),
        jax.ShapeDtypeStruct((NW, CNT_PAD), jnp.float32),
    ],
    scratch_types=[
        pltpu.VMEM((RPW + 16,), jnp.int32),
        pltpu.VMEM((RPW, D), jnp.float32),
        pltpu.VMEM((GP * D,), jnp.float32),
        pltpu.VMEM((GP * D,), jnp.float32),
        pltpu.VMEM((CNT_PAD,), jnp.float32),
    ],
)
def _readout(h_hbm, batch_hbm, osum, omax, ocnt, bidx, hrows, accs, accm, cnt):
    _readout_body(h_hbm, batch_hbm, osum, omax, ocnt, bidx, hrows, accs, accm,
                  cnt)


def _mlp_body(x_ref, p0_ref, p1_ref, w1_ref, b1_ref, w2_ref, b2_ref, o_ref):
    h = x_ref[...] + p0_ref[...] + p1_ref[...]
    a = jnp.maximum(
        jnp.dot(h, w1_ref[...], preferred_element_type=jnp.float32)
        + b1_ref[...], 0.0)
    o = jnp.maximum(
        jnp.dot(a, w2_ref[...], preferred_element_type=jnp.float32)
        + b2_ref[...], 0.0)
    o_ref[...] = o


_MLP_BLK = 1024


def _mlp(x, p0, p1, W1, b1, W2, b2):
    row_spec = pl.BlockSpec((_MLP_BLK, D), lambda i: (i, 0))
    full = lambda a, b: pl.BlockSpec((a, b), lambda i: (0, 0))
    return pl.pallas_call(
        _mlp_body,
        grid=(NR // _MLP_BLK,),
        in_specs=[
            row_spec, row_spec, row_spec,
            full(D, H), full(1, H), full(H, H), full(1, H),
        ],
        out_specs=row_spec,
        out_shape=jax.ShapeDtypeStruct((NR, D), jnp.float32),
    )(x, p0, p1, W1, b1, W2, b2)


def _final_body(s2, m2, s3, m3, s4, m4, cn, wl1, bl1, wl2, bl2, o_ref):
    c = jnp.sum(cn[...], axis=1, keepdims=True)
    rc = 1.0 / jnp.maximum(c, 1.0)

    def block(s_ref, m_ref):
        ssum = jnp.sum(s_ref[...], axis=0)
        mx = jnp.max(m_ref[...], axis=0)
        mx = jnp.where(jnp.isfinite(mx), mx, 0.0)
        return ssum * rc, mx, ssum

    mean2, mx2, ss2 = block(s2, m2)
    mean3, mx3, ss3 = block(s3, m3)
    mean4, mx4, ss4 = block(s4, m4)
    g = jnp.concatenate(
        [mean2 + mean3 + mean4, mx2 + mx3 + mx4, ss2 + ss3 + ss4], axis=-1)
    z = jnp.maximum(
        jnp.dot(g, wl1[...], preferred_element_type=jnp.float32) + bl1[...],
        0.0)
    logits = jnp.dot(z, wl2[...], preferred_element_type=jnp.float32) + bl2[...]
    mm = jnp.max(logits, axis=-1, keepdims=True)
    e = jnp.exp(logits - mm)
    o_ref[...] = (logits - mm) - jnp.log(jnp.sum(e, axis=-1, keepdims=True))


def _final(s2, m2, s3, m3, s4, m4, cnt_t, Wl1, bl1, Wl2, bl2):
    return pl.pallas_call(
        _final_body,
        out_shape=jax.ShapeDtypeStruct((G, C), jnp.float32),
    )(s2, m2, s3, m3, s4, m4, cnt_t, Wl1, bl1, Wl2, bl2)


def kernel(x, edge_index, batch, W1a, b1a, W2a, b2a, W1b, b1b, W2b, b2b, Wl1,
           bl1, Wl2, bl2):
    src = edge_index[0]
    dst = edge_index[1]
    pad_e = E_PAD - E
    src_p = jnp.concatenate([src, jnp.zeros((pad_e,), jnp.int32)])
    dummy_dst = DUMMY_ROW + (jnp.arange(pad_e, dtype=jnp.int32) % (NR - N))
    dst_p = jnp.concatenate([dst, dummy_dst])

    x_p = jnp.concatenate([x, jnp.zeros((NR - N, D), jnp.float32)])
    batch_p = jnp.concatenate(
        [batch, jnp.full((NR - N,), DUMMY_SEG, jnp.int32)])

    b1a_r = b1a.reshape(1, H)
    b2a_r = b2a.reshape(1, H)
    b1b_r = b1b.reshape(1, H)
    b2b_r = b2b.reshape(1, H)
    bl1_r = bl1.reshape(1, H)
    bl2_r = bl2.reshape(1, C)

    def conv(h, W1, b1, W2, b2):
        parts = _edge_agg(h, src_p, dst_p)
        return _mlp(h, parts[0], parts[1], W1, b1, W2, b2)

    h1 = conv(x_p, W1a, b1a_r, W2a, b2a_r)
    h2 = conv(h1, W1b, b1b_r, W2b, b2b_r)
    s2, m2, c2 = _readout(h2, batch_p)
    h3 = conv(h2, W1b, b1b_r, W2b, b2b_r)
    s3, m3, c3 = _readout(h3, batch_p)
    h4 = conv(h3, W1b, b1b_r, W2b, b2b_r)
    s4, m4, c4 = _readout(h4, batch_p)

    rs = lambda a: a.reshape(NW, GP, D)[:, :G, :]
    cnt_t = c2[:, :G].T
    return _final(rs(s2), rs(m2), rs(s3), rs(m3), rs(s4), rs(m4), cnt_t, Wl1,
                  bl1_r, Wl2, bl2_r)


# speedup vs baseline: 1.5181x; 1.4449x over previous
import functools

import jax
import jax.numpy as jnp
from jax import lax
from jax.experimental import pallas as pl
from jax.experimental.pallas import tpu as pltpu
from jax.experimental.pallas import tpu_sc as plsc

N = 10000
E = 320000
D = 128
H = 128
G = 64
C = 10

NC = 2
NS = 16
NW = NC * NS

NR = 10240
DUMMY_ROW = 10000
DUMMY_SEG = G

EK = 128
NCHUNK = 79
EPW = NCHUNK * EK
E_PAD = NW * EPW

RPT = NR // NS
RPW = NR // NW
GP = G + 1
CNT_PAD = 80

_mesh = plsc.VectorSubcoreMesh(core_axis_name="c", subcore_axis_name="s",
                               num_cores=NC, num_subcores=NS)


def _edge_agg_body(x_hbm, src_hbm, dst_hbm, out_hbm, sidx, didx, rows, zbuf,
                   aggs, gsem):
    c = lax.axis_index("c")
    s = lax.axis_index("s")
    w = c * NS + s

    def _zero(i, _):
        r = i // (D // 16)
        g = i % (D // 16)
        zbuf[r, pl.ds(g * 16, 16)] = jnp.zeros((16,), jnp.float32)
        return 0

    lax.fori_loop(0, 128 * (D // 16), _zero, 0)
    for k in range(RPT // 128):
        pltpu.sync_copy(zbuf, aggs.at[pl.ds(s * RPT + k * 128, 128)])
    plsc.subcore_barrier()

    def _step(j, _):
        base = w * EPW + j * EK
        pltpu.sync_copy(src_hbm.at[pl.ds(base, EK)], sidx)
        pltpu.sync_copy(dst_hbm.at[pl.ds(base, EK)], didx)
        pltpu.async_copy(x_hbm.at[sidx], rows, gsem).wait()
        pltpu.sync_copy(rows, aggs.at[didx], add=True)
        return 0

    lax.fori_loop(0, NCHUNK, _step, 0)

    plsc.subcore_barrier()
    pltpu.sync_copy(aggs.at[pl.ds(s * RPT, RPT)],
                    out_hbm.at[c, pl.ds(s * RPT, RPT)])


@functools.partial(
    pl.kernel,
    mesh=_mesh,
    out_type=jax.ShapeDtypeStruct((NC, NR, D), jnp.float32),
    scratch_types=[
        pltpu.VMEM((EK,), jnp.int32),
        pltpu.VMEM((EK,), jnp.int32),
        pltpu.VMEM((EK, D), jnp.float32),
        pltpu.VMEM((128, D), jnp.float32),
        pltpu.VMEM_SHARED((NR, D), jnp.float32),
        pltpu.SemaphoreType.DMA,
    ],
)
def _edge_agg(x_hbm, src_hbm, dst_hbm, out_hbm, sidx, didx, rows, zbuf, aggs,
              gsem):
    _edge_agg_body(x_hbm, src_hbm, dst_hbm, out_hbm, sidx, didx, rows, zbuf,
                   aggs, gsem)


def _readout_body(h_hbm, batch_hbm, osum, omax, ocnt, bidx, hrows, accs, accm,
                  cnt):
    c = lax.axis_index("c")
    s = lax.axis_index("s")
    w = c * NS + s

    def _init(i, _):
        accs[pl.ds(i * 16, 16)] = jnp.zeros((16,), jnp.float32)
        accm[pl.ds(i * 16, 16)] = jnp.full((16,), -jnp.inf, jnp.float32)
        return 0

    lax.fori_loop(0, GP * D // 16, _init, 0)
    for k in range(CNT_PAD // 16):
        cnt[pl.ds(k * 16, 16)] = jnp.zeros((16,), jnp.float32)

    pltpu.sync_copy(batch_hbm.at[pl.ds(w * RPW, RPW)], bidx.at[pl.ds(0, RPW)])
    pltpu.sync_copy(h_hbm.at[pl.ds(w * RPW, RPW)], hrows)

    e0 = jnp.where(lax.iota(jnp.int32, 16) == 0, 1.0, 0.0).astype(jnp.float32)

    def _row(i, _):
        b = bidx[pl.ds(i, 16)][0]
        base = b * D
        for g in range(D // 16):
            r = hrows[i, pl.ds(g * 16, 16)]
            off = base + g * 16
            accs[pl.ds(off, 16)] = accs[pl.ds(off, 16)] + r
            accm[pl.ds(off, 16)] = jnp.maximum(accm[pl.ds(off, 16)], r)
        cnt[pl.ds(b, 16)] = cnt[pl.ds(b, 16)] + e0
        return 0

    lax.fori_loop(0, RPW, _row, 0)
    pltpu.sync_copy(accs, osum.at[w])
    pltpu.sync_copy(accm, omax.at[w])
    pltpu.sync_copy(cnt, ocnt.at[w])


@functools.partial(
    pl.kernel,
    mesh=_mesh,
    out_type=[
        jax.ShapeDtypeStruct((NW, GP * D), jnp.float32),
        jax.ShapeDtypeStruct((NW, GP * D), jnp.float32),
        jax.ShapeDtypeStruct((NW, CNT_PAD), jnp.float32),
    ],
    scratch_types=[
        pltpu.VMEM((RPW + 16,), jnp.int32),
        pltpu.VMEM((RPW, D), jnp.float32),
        pltpu.VMEM((GP * D,), jnp.float32),
        pltpu.VMEM((GP * D,), jnp.float32),
        pltpu.VMEM((CNT_PAD,), jnp.float32),
    ],
)
def _readout(h_hbm, batch_hbm, osum, omax, ocnt, bidx, hrows, accs, accm, cnt):
    _readout_body(h_hbm, batch_hbm, osum, omax, ocnt, bidx, hrows, accs, accm,
                  cnt)


def _mlp_body(x_ref, p0_ref, p1_ref, w1_ref, b1_ref, w2_ref, b2_ref, o_ref):
    h = x_ref[...] + p0_ref[...] + p1_ref[...]
    a = jnp.maximum(
        jnp.dot(h, w1_ref[...], preferred_element_type=jnp.float32)
        + b1_ref[...], 0.0)
    o = jnp.maximum(
        jnp.dot(a, w2_ref[...], preferred_element_type=jnp.float32)
        + b2_ref[...], 0.0)
    o_ref[...] = o


_MLP_BLK = 1024


def _mlp(x, p0, p1, W1, b1, W2, b2):
    row_spec = pl.BlockSpec((_MLP_BLK, D), lambda i: (i, 0))
    full = lambda a, b: pl.BlockSpec((a, b), lambda i: (0, 0))
    return pl.pallas_call(
        _mlp_body,
        grid=(NR // _MLP_BLK,),
        in_specs=[
            row_spec, row_spec, row_spec,
            full(D, H), full(1, H), full(H, H), full(1, H),
        ],
        out_specs=row_spec,
        out_shape=jax.ShapeDtypeStruct((NR, D), jnp.float32),
    )(x, p0, p1, W1, b1, W2, b2)


def _final_body(s2, m2, s3, m3, s4, m4, cn, wl1, bl1, wl2, bl2, o_ref):
    c = jnp.sum(cn[...], axis=1, keepdims=True)
    rc = 1.0 / jnp.maximum(c, 1.0)

    def block(s_ref, m_ref):
        ssum = jnp.sum(s_ref[...], axis=0)
        mx = jnp.max(m_ref[...], axis=0)
        mx = jnp.where(jnp.isfinite(mx), mx, 0.0)
        return ssum * rc, mx, ssum

    mean2, mx2, ss2 = block(s2, m2)
    mean3, mx3, ss3 = block(s3, m3)
    mean4, mx4, ss4 = block(s4, m4)
    g = jnp.concatenate(
        [mean2 + mean3 + mean4, mx2 + mx3 + mx4, ss2 + ss3 + ss4], axis=-1)
    z = jnp.maximum(
        jnp.dot(g, wl1[...], preferred_element_type=jnp.float32) + bl1[...],
        0.0)
    logits = jnp.dot(z, wl2[...], preferred_element_type=jnp.float32) + bl2[...]
    mm = jnp.max(logits, axis=-1, keepdims=True)
    e = jnp.exp(logits - mm)
    o_ref[...] = (logits - mm) - jnp.log(jnp.sum(e, axis=-1, keepdims=True))


def _final(s2, m2, s3, m3, s4, m4, cnt_t, Wl1, bl1, Wl2, bl2):
    return pl.pallas_call(
        _final_body,
        out_shape=jax.ShapeDtypeStruct((G, C), jnp.float32),
    )(s2, m2, s3, m3, s4, m4, cnt_t, Wl1, bl1, Wl2, bl2)


def kernel(x, edge_index, batch, W1a, b1a, W2a, b2a, W1b, b1b, W2b, b2b, Wl1,
           bl1, Wl2, bl2):
    src = edge_index[0]
    dst = edge_index[1]
    pad_e = E_PAD - E
    src_p = jnp.concatenate([src, jnp.zeros((pad_e,), jnp.int32)])
    dst_p = jnp.concatenate([dst, jnp.full((pad_e,), DUMMY_ROW, jnp.int32)])

    x_p = jnp.concatenate([x, jnp.zeros((NR - N, D), jnp.float32)])
    batch_p = jnp.concatenate(
        [batch, jnp.full((NR - N,), DUMMY_SEG, jnp.int32)])

    b1a_r = b1a.reshape(1, H)
    b2a_r = b2a.reshape(1, H)
    b1b_r = b1b.reshape(1, H)
    b2b_r = b2b.reshape(1, H)
    bl1_r = bl1.reshape(1, H)
    bl2_r = bl2.reshape(1, C)

    def conv(h, W1, b1, W2, b2):
        parts = _edge_agg(h, src_p, dst_p)
        return _mlp(h, parts[0], parts[1], W1, b1, W2, b2)

    h1 = conv(x_p, W1a, b1a_r, W2a, b2a_r)
    h2 = conv(h1, W1b, b1b_r, W2b, b2b_r)
    s2, m2, c2 = _readout(h2, batch_p)
    h3 = conv(h2, W1b, b1b_r, W2b, b2b_r)
    s3, m3, c3 = _readout(h3, batch_p)
    h4 = conv(h3, W1b, b1b_r, W2b, b2b_r)
    s4, m4, c4 = _readout(h4, batch_p)

    rs = lambda a: a.reshape(NW, GP, D)[:, :G, :]
    cnt_t = c2[:, :G].T
    return _final(rs(s2), rs(m2), rs(s3), rs(m3), rs(s4), rs(m4), cnt_t, Wl1,
                  bl1_r, Wl2, bl2_r)
